# baseline (speedup 1.0000x reference)
import functools
import jax
import jax.numpy as jnp
from jax import lax
from jax.experimental import pallas as pl
from jax.experimental.pallas import tpu as pltpu
from jax.experimental.pallas import tpu_sc as plsc

N = 10000
E = 320000
D_IN = 128
D_HID = 64
N_GRAPHS = 20
N_CLUSTERS = 25

NC = 2
NS = 16
NW = NC * NS
CHUNK = 50
NCH = E // (NW * CHUNK)
IB = 40
NIB = NCH // IB
ROWS_PER_TILE = 640
LAST_ROWS = N - (NS - 1) * ROWS_PER_TILE
DW = 128
NBUF = 4

_mesh = plsc.VectorSubcoreMesh(core_axis_name="c", subcore_axis_name="s")


N_PAD = NS * ROWS_PER_TILE


@functools.partial(
    pl.kernel,
    mesh=_mesh,
    out_type=jax.ShapeDtypeStruct((NC, 1, N_PAD), jnp.float32),
    scratch_types=[
        pltpu.VMEM((NCH, CHUNK), jnp.int32),
        pltpu.VMEM((128,), jnp.float32),
        pltpu.VMEM((ROWS_PER_TILE,), jnp.float32),
        pltpu.VMEM_SHARED((N_PAD,), jnp.float32),
        pltpu.SemaphoreType.DMA,
    ],
)
def _sc_degree(edges_hbm, out_hbm, dst_v, ones_v, zero_v, dacc, dsem):
    c = lax.axis_index("c")
    s = lax.axis_index("s")
    wid = c * NS + s
    for i in range(128 // 16):
        ones_v[pl.ds(16 * i, 16)] = jnp.ones((16,), jnp.float32)
    for i in range(ROWS_PER_TILE // 16):
        zero_v[pl.ds(16 * i, 16)] = jnp.zeros((16,), jnp.float32)

    pltpu.sync_copy(zero_v, dacc.at[pl.ds(ROWS_PER_TILE * s, ROWS_PER_TILE)])
    plsc.subcore_barrier()
    pltpu.sync_copy(edges_hbm.at[1, wid], dst_v)

    def fire(j, carry):
        pltpu.async_copy(ones_v.at[pl.ds(0, CHUNK)], dacc.at[dst_v.at[j]],
                         dsem, add=True)
        return carry

    lax.fori_loop(0, NCH, fire, 0)

    def drain(j, carry):
        pltpu.make_async_copy(ones_v.at[pl.ds(0, CHUNK)],
                              dacc.at[dst_v.at[0]], dsem).wait()
        return carry

    lax.fori_loop(0, NCH, drain, 0)
    plsc.subcore_barrier()
    pltpu.sync_copy(dacc.at[pl.ds(ROWS_PER_TILE * s, ROWS_PER_TILE)],
                    out_hbm.at[c, 0, pl.ds(ROWS_PER_TILE * s, ROWS_PER_TILE)])


@functools.partial(
    pl.kernel,
    mesh=_mesh,
    out_type=jax.ShapeDtypeStruct((NC, N, DW), jnp.float32),
    scratch_types=[
        pltpu.VMEM((IB, CHUNK), jnp.int32),
        pltpu.VMEM((IB, CHUNK), jnp.int32),
        pltpu.VMEM((NBUF, CHUNK, DW), jnp.float32),
        pltpu.VMEM_SHARED((N, DW), jnp.float32),
        pltpu.SemaphoreType.DMA,
        pltpu.SemaphoreType.DMA,
    ],
)
def _sc_agg(y_hbm, edges_hbm, out_hbm, src_v, dst_v, rows_v, acc,
            gsem, ssem):
    c = lax.axis_index("c")
    s = lax.axis_index("s")
    wid = c * NS + s

    @pl.when(s < NS - 1)
    def _():
        pltpu.async_copy(y_hbm.at[pl.ds(ROWS_PER_TILE * s, ROWS_PER_TILE)],
                         acc.at[pl.ds(ROWS_PER_TILE * s, ROWS_PER_TILE)], gsem)

    @pl.when(s == NS - 1)
    def _():
        pltpu.async_copy(y_hbm.at[pl.ds(ROWS_PER_TILE * (NS - 1), LAST_ROWS)],
                         acc.at[pl.ds(ROWS_PER_TILE * (NS - 1), LAST_ROWS)],
                         gsem)

    pltpu.sync_copy(edges_hbm.at[0, wid, pl.ds(0, IB)], src_v)
    pltpu.sync_copy(edges_hbm.at[1, wid, pl.ds(0, IB)], dst_v)

    @pl.when(s < NS - 1)
    def _():
        pltpu.make_async_copy(
            y_hbm.at[pl.ds(ROWS_PER_TILE * s, ROWS_PER_TILE)],
            acc.at[pl.ds(ROWS_PER_TILE * s, ROWS_PER_TILE)], gsem).wait()

    @pl.when(s == NS - 1)
    def _():
        pltpu.make_async_copy(
            y_hbm.at[pl.ds(ROWS_PER_TILE * (NS - 1), LAST_ROWS)],
            acc.at[pl.ds(ROWS_PER_TILE * (NS - 1), LAST_ROWS)], gsem).wait()

    plsc.subcore_barrier()

    def ib_body(ib, carry):
        @pl.when(ib > 0)
        def _():
            pltpu.sync_copy(edges_hbm.at[0, wid, pl.ds(IB * ib, IB)], src_v)
            pltpu.sync_copy(edges_hbm.at[1, wid, pl.ds(IB * ib, IB)], dst_v)

        for b in range(NBUF - 1):
            pltpu.async_copy(y_hbm.at[src_v.at[b]], rows_v.at[b], gsem)

        def chunk_body(j, carry2):
            b = lax.rem(j, NBUF)
            pltpu.make_async_copy(y_hbm.at[src_v.at[j]], rows_v.at[b],
                                  gsem).wait()
            pltpu.async_copy(rows_v.at[b], acc.at[dst_v.at[j]], ssem,
                             add=True)

            @pl.when(j > 0)
            def _():
                pltpu.make_async_copy(rows_v.at[b], acc.at[dst_v.at[0]],
                                      ssem).wait()

            @pl.when(j + NBUF - 1 < IB)
            def _():
                bnext = lax.rem(j + NBUF - 1, NBUF)
                pltpu.async_copy(y_hbm.at[src_v.at[j + NBUF - 1]],
                                 rows_v.at[bnext], gsem)

            return carry2

        lax.fori_loop(0, IB, chunk_body, 0)
        pltpu.make_async_copy(rows_v.at[0], acc.at[dst_v.at[0]], ssem).wait()
        return carry

    lax.fori_loop(0, NIB, ib_body, 0)
    plsc.subcore_barrier()

    @pl.when(s < NS - 1)
    def _():
        pltpu.sync_copy(acc.at[pl.ds(ROWS_PER_TILE * s, ROWS_PER_TILE)],
                        out_hbm.at[c, pl.ds(ROWS_PER_TILE * s, ROWS_PER_TILE)])

    @pl.when(s == NS - 1)
    def _():
        pltpu.sync_copy(acc.at[pl.ds(ROWS_PER_TILE * (NS - 1), LAST_ROWS)],
                        out_hbm.at[c, pl.ds(ROWS_PER_TILE * (NS - 1), LAST_ROWS)])


_BLK = 1000
_GRID = N // _BLK


def _tc0_body(x_ref, w1_ref, h1_ref):
    h1_ref[...] = jnp.dot(x_ref[...], w1_ref[...],
                          preferred_element_type=jnp.float32)


def _tc0(x, w1):
    return pl.pallas_call(
        _tc0_body,
        grid=(_GRID,),
        in_specs=[
            pl.BlockSpec((_BLK, D_IN), lambda i: (i, 0)),
            pl.BlockSpec((D_IN, D_HID), lambda i: (0, 0)),
        ],
        out_specs=pl.BlockSpec((_BLK, D_HID), lambda i: (i, 0)),
        out_shape=jax.ShapeDtypeStruct((N, D_HID), jnp.float32),
    )(x, w1)


def _tc1_body(h1_ref, d0_ref, d1_ref, h1s_ref, dinv_ref):
    dinv = lax.rsqrt(d0_ref[...] + d1_ref[...] + 1.0)
    h1s_ref[:, 0:D_HID] = h1_ref[...] * dinv
    h1s_ref[:, D_HID:DW] = jnp.zeros((_BLK, DW - D_HID), jnp.float32)
    dinv_ref[...] = dinv


def _tc1(h1, d0, d1):
    return pl.pallas_call(
        _tc1_body,
        grid=(_GRID,),
        in_specs=[
            pl.BlockSpec((_BLK, D_HID), lambda i: (i, 0)),
            pl.BlockSpec((_BLK, 1), lambda i: (i, 0)),
            pl.BlockSpec((_BLK, 1), lambda i: (i, 0)),
        ],
        out_specs=[
            pl.BlockSpec((_BLK, DW), lambda i: (i, 0)),
            pl.BlockSpec((_BLK, 1), lambda i: (i, 0)),
        ],
        out_shape=[
            jax.ShapeDtypeStruct((N, DW), jnp.float32),
            jax.ShapeDtypeStruct((N, 1), jnp.float32),
        ],
    )(h1, d0, d1)


def _tc2_body(p0_ref, p1_ref, h1s_ref, dinv_ref, b1_ref, w2_ref, h2s_ref):
    dinv = dinv_ref[...]
    t = (p0_ref[0, :, 0:D_HID] + p1_ref[0, :, 0:D_HID]
         - h1s_ref[:, 0:D_HID])
    z1 = jnp.maximum(t * dinv + b1_ref[...], 0.0)
    h2 = jnp.dot(z1, w2_ref[...], preferred_element_type=jnp.float32)
    h2s_ref[:, 0:D_HID] = h2 * dinv
    h2s_ref[:, D_HID:DW] = jnp.zeros((_BLK, DW - D_HID), jnp.float32)


def _tc2(p, h1s, dinv, b1, w2):
    return pl.pallas_call(
        _tc2_body,
        grid=(_GRID,),
        in_specs=[
            pl.BlockSpec((1, _BLK, DW), lambda i: (0, i, 0)),
            pl.BlockSpec((1, _BLK, DW), lambda i: (1, i, 0)),
            pl.BlockSpec((_BLK, DW), lambda i: (i, 0)),
            pl.BlockSpec((_BLK, 1), lambda i: (i, 0)),
            pl.BlockSpec((1, D_HID), lambda i: (0, 0)),
            pl.BlockSpec((D_HID, D_HID), lambda i: (0, 0)),
        ],
        out_specs=pl.BlockSpec((_BLK, DW), lambda i: (i, 0)),
        out_shape=jax.ShapeDtypeStruct((N, DW), jnp.float32),
    )(p, p, h1s, dinv, b1, w2)


_BPAD = 24


def _tc3_body(q0_ref, q1_ref, h2s_ref, dinv_ref, b2_ref, batch_ref,
              out_ref, acc_ref):
    i = pl.program_id(0)
    z2 = jnp.maximum(
        (q0_ref[0, :, 0:D_HID] + q1_ref[0, :, 0:D_HID]
         - h2s_ref[:, 0:D_HID]) * dinv_ref[...] + b2_ref[...], 0.0)
    bb = batch_ref[0]
    iot = lax.broadcasted_iota(jnp.int32, (_BPAD, _BLK), 0)
    onehot = (iot == bb).astype(jnp.float32)
    part = jnp.dot(onehot, z2, preferred_element_type=jnp.float32)

    @pl.when(i == 0)
    def _():
        acc_ref[...] = part

    @pl.when(i > 0)
    def _():
        acc_ref[...] = acc_ref[...] + part

    @pl.when(i == _GRID - 1)
    def _():
        out_ref[...] = acc_ref[0:N_GRAPHS, :] * (1.0 / N_CLUSTERS)


def _tc3(q, h2s, dinv, b2, batch3d):
    return pl.pallas_call(
        _tc3_body,
        grid=(_GRID,),
        in_specs=[
            pl.BlockSpec((1, _BLK, DW), lambda i: (0, i, 0)),
            pl.BlockSpec((1, _BLK, DW), lambda i: (1, i, 0)),
            pl.BlockSpec((_BLK, DW), lambda i: (i, 0)),
            pl.BlockSpec((_BLK, 1), lambda i: (i, 0)),
            pl.BlockSpec((1, D_HID), lambda i: (0, 0)),
            pl.BlockSpec((1, 1, _BLK), lambda i: (i, 0, 0)),
        ],
        out_specs=pl.BlockSpec((N_GRAPHS, D_HID), lambda i: (0, 0)),
        out_shape=jax.ShapeDtypeStruct((N_GRAPHS, D_HID), jnp.float32),
        scratch_shapes=[pltpu.VMEM((_BPAD, D_HID), jnp.float32)],
    )(q, q, h2s, dinv, b2, batch3d)


def kernel(x, edge_index, batch, W1, b1, W2, b2, W3, b3, W4, b4):
    edges = edge_index.reshape(2, NW, NCH, CHUNK)

    h1 = _tc0(x, W1)
    deg = _sc_degree(edges)
    d0 = deg[0, 0, :N].reshape(N, 1)
    d1 = deg[1, 0, :N].reshape(N, 1)
    h1s, dinv = _tc1(h1, d0, d1)

    p = _sc_agg(h1s, edges)
    h2s = _tc2(p, h1s, dinv, b1.reshape(1, D_HID), W2)

    q = _sc_agg(h2s, edges)
    out = _tc3(q, h2s, dinv, b2.reshape(1, D_HID),
               batch.reshape(_GRID, 1, _BLK))
    return out


# speedup vs baseline: 1.0169x; 1.0169x over previous
import functools
import jax
import jax.numpy as jnp
from jax import lax
from jax.experimental import pallas as pl
from jax.experimental.pallas import tpu as pltpu
from jax.experimental.pallas import tpu_sc as plsc

N = 10000
E = 320000
D_IN = 128
D_HID = 64
N_GRAPHS = 20
N_CLUSTERS = 25

NC = 2
NS = 16
NW = NC * NS
CHUNK = 50
NCH = E // (NW * CHUNK)
IB = 40
NIB = NCH // IB
ROWS_PER_TILE = 640
LAST_ROWS = N - (NS - 1) * ROWS_PER_TILE
DW = 128
NBUF = 5

_mesh = plsc.VectorSubcoreMesh(core_axis_name="c", subcore_axis_name="s")


N_PAD = NS * ROWS_PER_TILE


@functools.partial(
    pl.kernel,
    mesh=_mesh,
    out_type=jax.ShapeDtypeStruct((NC, 1, N_PAD), jnp.float32),
    scratch_types=[
        pltpu.VMEM((NCH, CHUNK), jnp.int32),
        pltpu.VMEM((128,), jnp.float32),
        pltpu.VMEM((ROWS_PER_TILE,), jnp.float32),
        pltpu.VMEM_SHARED((N_PAD,), jnp.float32),
        pltpu.SemaphoreType.DMA,
    ],
)
def _sc_degree(edges_hbm, out_hbm, dst_v, ones_v, zero_v, dacc, dsem):
    c = lax.axis_index("c")
    s = lax.axis_index("s")
    wid = c * NS + s
    for i in range(128 // 16):
        ones_v[pl.ds(16 * i, 16)] = jnp.ones((16,), jnp.float32)
    for i in range(ROWS_PER_TILE // 16):
        zero_v[pl.ds(16 * i, 16)] = jnp.zeros((16,), jnp.float32)

    pltpu.sync_copy(zero_v, dacc.at[pl.ds(ROWS_PER_TILE * s, ROWS_PER_TILE)])
    plsc.subcore_barrier()
    pltpu.sync_copy(edges_hbm.at[1, wid], dst_v)

    def fire(j, carry):
        pltpu.async_copy(ones_v.at[pl.ds(0, CHUNK)], dacc.at[dst_v.at[j]],
                         dsem, add=True)
        return carry

    lax.fori_loop(0, NCH, fire, 0)

    def drain(j, carry):
        pltpu.make_async_copy(ones_v.at[pl.ds(0, CHUNK)],
                              dacc.at[dst_v.at[0]], dsem).wait()
        return carry

    lax.fori_loop(0, NCH, drain, 0)
    plsc.subcore_barrier()
    pltpu.sync_copy(dacc.at[pl.ds(ROWS_PER_TILE * s, ROWS_PER_TILE)],
                    out_hbm.at[c, 0, pl.ds(ROWS_PER_TILE * s, ROWS_PER_TILE)])


@functools.partial(
    pl.kernel,
    mesh=_mesh,
    out_type=jax.ShapeDtypeStruct((NC, N, DW), jnp.float32),
    scratch_types=[
        pltpu.VMEM((IB, CHUNK), jnp.int32),
        pltpu.VMEM((IB, CHUNK), jnp.int32),
        pltpu.VMEM((NBUF, CHUNK, DW), jnp.float32),
        pltpu.VMEM_SHARED((N, DW), jnp.float32),
        pltpu.SemaphoreType.DMA,
        pltpu.SemaphoreType.DMA,
    ],
)
def _sc_agg(y_hbm, edges_hbm, out_hbm, src_v, dst_v, rows_v, acc,
            gsem, ssem):
    c = lax.axis_index("c")
    s = lax.axis_index("s")
    wid = c * NS + s

    @pl.when(s < NS - 1)
    def _():
        pltpu.async_copy(y_hbm.at[pl.ds(ROWS_PER_TILE * s, ROWS_PER_TILE)],
                         acc.at[pl.ds(ROWS_PER_TILE * s, ROWS_PER_TILE)], gsem)

    @pl.when(s == NS - 1)
    def _():
        pltpu.async_copy(y_hbm.at[pl.ds(ROWS_PER_TILE * (NS - 1), LAST_ROWS)],
                         acc.at[pl.ds(ROWS_PER_TILE * (NS - 1), LAST_ROWS)],
                         gsem)

    pltpu.sync_copy(edges_hbm.at[0, wid, pl.ds(0, IB)], src_v)
    pltpu.sync_copy(edges_hbm.at[1, wid, pl.ds(0, IB)], dst_v)

    @pl.when(s < NS - 1)
    def _():
        pltpu.make_async_copy(
            y_hbm.at[pl.ds(ROWS_PER_TILE * s, ROWS_PER_TILE)],
            acc.at[pl.ds(ROWS_PER_TILE * s, ROWS_PER_TILE)], gsem).wait()

    @pl.when(s == NS - 1)
    def _():
        pltpu.make_async_copy(
            y_hbm.at[pl.ds(ROWS_PER_TILE * (NS - 1), LAST_ROWS)],
            acc.at[pl.ds(ROWS_PER_TILE * (NS - 1), LAST_ROWS)], gsem).wait()

    plsc.subcore_barrier()

    def ib_body(ib, carry):
        @pl.when(ib > 0)
        def _():
            pltpu.sync_copy(edges_hbm.at[0, wid, pl.ds(IB * ib, IB)], src_v)
            pltpu.sync_copy(edges_hbm.at[1, wid, pl.ds(IB * ib, IB)], dst_v)

        for b in range(NBUF - 1):
            pltpu.async_copy(y_hbm.at[src_v.at[b]], rows_v.at[b], gsem)

        def chunk_body(j, carry2):
            b = lax.rem(j, NBUF)
            pltpu.make_async_copy(y_hbm.at[src_v.at[j]], rows_v.at[b],
                                  gsem).wait()
            pltpu.async_copy(rows_v.at[b], acc.at[dst_v.at[j]], ssem,
                             add=True)

            @pl.when(j > 0)
            def _():
                pltpu.make_async_copy(rows_v.at[b], acc.at[dst_v.at[0]],
                                      ssem).wait()

            @pl.when(j + NBUF - 1 < IB)
            def _():
                bnext = lax.rem(j + NBUF - 1, NBUF)
                pltpu.async_copy(y_hbm.at[src_v.at[j + NBUF - 1]],
                                 rows_v.at[bnext], gsem)

            return carry2

        lax.fori_loop(0, IB, chunk_body, 0)
        pltpu.make_async_copy(rows_v.at[0], acc.at[dst_v.at[0]], ssem).wait()
        return carry

    lax.fori_loop(0, NIB, ib_body, 0)
    plsc.subcore_barrier()

    @pl.when(s < NS - 1)
    def _():
        pltpu.sync_copy(acc.at[pl.ds(ROWS_PER_TILE * s, ROWS_PER_TILE)],
                        out_hbm.at[c, pl.ds(ROWS_PER_TILE * s, ROWS_PER_TILE)])

    @pl.when(s == NS - 1)
    def _():
        pltpu.sync_copy(acc.at[pl.ds(ROWS_PER_TILE * (NS - 1), LAST_ROWS)],
                        out_hbm.at[c, pl.ds(ROWS_PER_TILE * (NS - 1), LAST_ROWS)])


_BLK = 1000
_GRID = N // _BLK


def _tc0_body(x_ref, w1_ref, h1_ref):
    h1_ref[...] = jnp.dot(x_ref[...], w1_ref[...],
                          preferred_element_type=jnp.float32)


def _tc0(x, w1):
    return pl.pallas_call(
        _tc0_body,
        grid=(_GRID,),
        in_specs=[
            pl.BlockSpec((_BLK, D_IN), lambda i: (i, 0)),
            pl.BlockSpec((D_IN, D_HID), lambda i: (0, 0)),
        ],
        out_specs=pl.BlockSpec((_BLK, D_HID), lambda i: (i, 0)),
        out_shape=jax.ShapeDtypeStruct((N, D_HID), jnp.float32),
    )(x, w1)


def _tc1_body(h1_ref, d0_ref, d1_ref, h1s_ref, dinv_ref):
    dinv = lax.rsqrt(d0_ref[...] + d1_ref[...] + 1.0)
    h1s_ref[:, 0:D_HID] = h1_ref[...] * dinv
    h1s_ref[:, D_HID:DW] = jnp.zeros((_BLK, DW - D_HID), jnp.float32)
    dinv_ref[...] = dinv


def _tc1(h1, d0, d1):
    return pl.pallas_call(
        _tc1_body,
        grid=(_GRID,),
        in_specs=[
            pl.BlockSpec((_BLK, D_HID), lambda i: (i, 0)),
            pl.BlockSpec((_BLK, 1), lambda i: (i, 0)),
            pl.BlockSpec((_BLK, 1), lambda i: (i, 0)),
        ],
        out_specs=[
            pl.BlockSpec((_BLK, DW), lambda i: (i, 0)),
            pl.BlockSpec((_BLK, 1), lambda i: (i, 0)),
        ],
        out_shape=[
            jax.ShapeDtypeStruct((N, DW), jnp.float32),
            jax.ShapeDtypeStruct((N, 1), jnp.float32),
        ],
    )(h1, d0, d1)


def _tc2_body(p0_ref, p1_ref, h1s_ref, dinv_ref, b1_ref, w2_ref, h2s_ref):
    dinv = dinv_ref[...]
    t = (p0_ref[0, :, 0:D_HID] + p1_ref[0, :, 0:D_HID]
         - h1s_ref[:, 0:D_HID])
    z1 = jnp.maximum(t * dinv + b1_ref[...], 0.0)
    h2 = jnp.dot(z1, w2_ref[...], preferred_element_type=jnp.float32)
    h2s_ref[:, 0:D_HID] = h2 * dinv
    h2s_ref[:, D_HID:DW] = jnp.zeros((_BLK, DW - D_HID), jnp.float32)


def _tc2(p, h1s, dinv, b1, w2):
    return pl.pallas_call(
        _tc2_body,
        grid=(_GRID,),
        in_specs=[
            pl.BlockSpec((1, _BLK, DW), lambda i: (0, i, 0)),
            pl.BlockSpec((1, _BLK, DW), lambda i: (1, i, 0)),
            pl.BlockSpec((_BLK, DW), lambda i: (i, 0)),
            pl.BlockSpec((_BLK, 1), lambda i: (i, 0)),
            pl.BlockSpec((1, D_HID), lambda i: (0, 0)),
            pl.BlockSpec((D_HID, D_HID), lambda i: (0, 0)),
        ],
        out_specs=pl.BlockSpec((_BLK, DW), lambda i: (i, 0)),
        out_shape=jax.ShapeDtypeStruct((N, DW), jnp.float32),
    )(p, p, h1s, dinv, b1, w2)


_BPAD = 24


def _tc3_body(q0_ref, q1_ref, h2s_ref, dinv_ref, b2_ref, batch_ref,
              out_ref, acc_ref):
    i = pl.program_id(0)
    z2 = jnp.maximum(
        (q0_ref[0, :, 0:D_HID] + q1_ref[0, :, 0:D_HID]
         - h2s_ref[:, 0:D_HID]) * dinv_ref[...] + b2_ref[...], 0.0)
    bb = batch_ref[0]
    iot = lax.broadcasted_iota(jnp.int32, (_BPAD, _BLK), 0)
    onehot = (iot == bb).astype(jnp.float32)
    part = jnp.dot(onehot, z2, preferred_element_type=jnp.float32)

    @pl.when(i == 0)
    def _():
        acc_ref[...] = part

    @pl.when(i > 0)
    def _():
        acc_ref[...] = acc_ref[...] + part

    @pl.when(i == _GRID - 1)
    def _():
        out_ref[...] = acc_ref[0:N_GRAPHS, :] * (1.0 / N_CLUSTERS)


def _tc3(q, h2s, dinv, b2, batch3d):
    return pl.pallas_call(
        _tc3_body,
        grid=(_GRID,),
        in_specs=[
            pl.BlockSpec((1, _BLK, DW), lambda i: (0, i, 0)),
            pl.BlockSpec((1, _BLK, DW), lambda i: (1, i, 0)),
            pl.BlockSpec((_BLK, DW), lambda i: (i, 0)),
            pl.BlockSpec((_BLK, 1), lambda i: (i, 0)),
            pl.BlockSpec((1, D_HID), lambda i: (0, 0)),
            pl.BlockSpec((1, 1, _BLK), lambda i: (i, 0, 0)),
        ],
        out_specs=pl.BlockSpec((N_GRAPHS, D_HID), lambda i: (0, 0)),
        out_shape=jax.ShapeDtypeStruct((N_GRAPHS, D_HID), jnp.float32),
        scratch_shapes=[pltpu.VMEM((_BPAD, D_HID), jnp.float32)],
    )(q, q, h2s, dinv, b2, batch3d)


def kernel(x, edge_index, batch, W1, b1, W2, b2, W3, b3, W4, b4):
    edges = edge_index.reshape(2, NW, NCH, CHUNK)

    h1 = _tc0(x, W1)
    deg = _sc_degree(edges)
    d0 = deg[0, 0, :N].reshape(N, 1)
    d1 = deg[1, 0, :N].reshape(N, 1)
    h1s, dinv = _tc1(h1, d0, d1)

    p = _sc_agg(h1s, edges)
    h2s = _tc2(p, h1s, dinv, b1.reshape(1, D_HID), W2)

    q = _sc_agg(h2s, edges)
    out = _tc3(q, h2s, dinv, b2.reshape(1, D_HID),
               batch.reshape(_GRID, 1, _BLK))
    return out


# speedup vs baseline: 1.1590x; 1.1398x over previous
import functools
import jax
import jax.numpy as jnp
from jax import lax
from jax.experimental import pallas as pl
from jax.experimental.pallas import tpu as pltpu
from jax.experimental.pallas import tpu_sc as plsc

N = 10000
E = 320000
D_IN = 128
D_HID = 64
N_GRAPHS = 20
N_CLUSTERS = 25

NC = 2
NS = 16
NW = NC * NS
CHUNK = 50
NCH = E // (NW * CHUNK)
IB = 40
NIB = NCH // IB
ROWS_PER_TILE = 640
LAST_ROWS = N - (NS - 1) * ROWS_PER_TILE
DW = 64
NBUF = 5

_mesh = plsc.VectorSubcoreMesh(core_axis_name="c", subcore_axis_name="s")
_sc_params = pltpu.CompilerParams(use_tc_tiling_on_sc=False)


N_PAD = NS * ROWS_PER_TILE


@functools.partial(
    pl.kernel,
    mesh=_mesh,
    compiler_params=_sc_params,
    out_type=jax.ShapeDtypeStruct((NC, 1, N_PAD), jnp.float32),
    scratch_types=[
        pltpu.VMEM((NCH, CHUNK), jnp.int32),
        pltpu.VMEM((128,), jnp.float32),
        pltpu.VMEM((ROWS_PER_TILE,), jnp.float32),
        pltpu.VMEM_SHARED((N_PAD,), jnp.float32),
        pltpu.SemaphoreType.DMA,
    ],
)
def _sc_degree(edges_hbm, out_hbm, dst_v, ones_v, zero_v, dacc, dsem):
    c = lax.axis_index("c")
    s = lax.axis_index("s")
    wid = c * NS + s
    for i in range(128 // 16):
        ones_v[pl.ds(16 * i, 16)] = jnp.ones((16,), jnp.float32)
    for i in range(ROWS_PER_TILE // 16):
        zero_v[pl.ds(16 * i, 16)] = jnp.zeros((16,), jnp.float32)

    pltpu.sync_copy(zero_v, dacc.at[pl.ds(ROWS_PER_TILE * s, ROWS_PER_TILE)])
    plsc.subcore_barrier()
    pltpu.sync_copy(edges_hbm.at[1, wid], dst_v)

    def fire(j, carry):
        pltpu.async_copy(ones_v.at[pl.ds(0, CHUNK)], dacc.at[dst_v.at[j]],
                         dsem, add=True)
        return carry

    lax.fori_loop(0, NCH, fire, 0)

    def drain(j, carry):
        pltpu.make_async_copy(ones_v.at[pl.ds(0, CHUNK)],
                              dacc.at[dst_v.at[0]], dsem).wait()
        return carry

    lax.fori_loop(0, NCH, drain, 0)
    plsc.subcore_barrier()
    pltpu.sync_copy(dacc.at[pl.ds(ROWS_PER_TILE * s, ROWS_PER_TILE)],
                    out_hbm.at[c, 0, pl.ds(ROWS_PER_TILE * s, ROWS_PER_TILE)])


@functools.partial(
    pl.kernel,
    mesh=_mesh,
    compiler_params=_sc_params,
    out_type=jax.ShapeDtypeStruct((NC, N, DW), jnp.float32),
    scratch_types=[
        pltpu.VMEM((IB, CHUNK), jnp.int32),
        pltpu.VMEM((IB, CHUNK), jnp.int32),
        pltpu.VMEM((NBUF, CHUNK, DW), jnp.float32),
        pltpu.VMEM_SHARED((N, DW), jnp.float32),
        pltpu.SemaphoreType.DMA,
        pltpu.SemaphoreType.DMA,
    ],
)
def _sc_agg(y_hbm, edges_hbm, out_hbm, src_v, dst_v, rows_v, acc,
            gsem, ssem):
    c = lax.axis_index("c")
    s = lax.axis_index("s")
    wid = c * NS + s

    @pl.when(s < NS - 1)
    def _():
        pltpu.async_copy(y_hbm.at[pl.ds(ROWS_PER_TILE * s, ROWS_PER_TILE)],
                         acc.at[pl.ds(ROWS_PER_TILE * s, ROWS_PER_TILE)], gsem)

    @pl.when(s == NS - 1)
    def _():
        pltpu.async_copy(y_hbm.at[pl.ds(ROWS_PER_TILE * (NS - 1), LAST_ROWS)],
                         acc.at[pl.ds(ROWS_PER_TILE * (NS - 1), LAST_ROWS)],
                         gsem)

    pltpu.sync_copy(edges_hbm.at[0, wid, pl.ds(0, IB)], src_v)
    pltpu.sync_copy(edges_hbm.at[1, wid, pl.ds(0, IB)], dst_v)

    @pl.when(s < NS - 1)
    def _():
        pltpu.make_async_copy(
            y_hbm.at[pl.ds(ROWS_PER_TILE * s, ROWS_PER_TILE)],
            acc.at[pl.ds(ROWS_PER_TILE * s, ROWS_PER_TILE)], gsem).wait()

    @pl.when(s == NS - 1)
    def _():
        pltpu.make_async_copy(
            y_hbm.at[pl.ds(ROWS_PER_TILE * (NS - 1), LAST_ROWS)],
            acc.at[pl.ds(ROWS_PER_TILE * (NS - 1), LAST_ROWS)], gsem).wait()

    plsc.subcore_barrier()

    def ib_body(ib, carry):
        @pl.when(ib > 0)
        def _():
            pltpu.sync_copy(edges_hbm.at[0, wid, pl.ds(IB * ib, IB)], src_v)
            pltpu.sync_copy(edges_hbm.at[1, wid, pl.ds(IB * ib, IB)], dst_v)

        for b in range(NBUF - 1):
            pltpu.async_copy(y_hbm.at[src_v.at[b]], rows_v.at[b], gsem)

        def chunk_body(j, carry2):
            b = lax.rem(j, NBUF)
            pltpu.make_async_copy(y_hbm.at[src_v.at[j]], rows_v.at[b],
                                  gsem).wait()
            pltpu.async_copy(rows_v.at[b], acc.at[dst_v.at[j]], ssem,
                             add=True)

            @pl.when(j > 0)
            def _():
                pltpu.make_async_copy(rows_v.at[b], acc.at[dst_v.at[0]],
                                      ssem).wait()

            @pl.when(j + NBUF - 1 < IB)
            def _():
                bnext = lax.rem(j + NBUF - 1, NBUF)
                pltpu.async_copy(y_hbm.at[src_v.at[j + NBUF - 1]],
                                 rows_v.at[bnext], gsem)

            return carry2

        lax.fori_loop(0, IB, chunk_body, 0)
        pltpu.make_async_copy(rows_v.at[0], acc.at[dst_v.at[0]], ssem).wait()
        return carry

    lax.fori_loop(0, NIB, ib_body, 0)
    plsc.subcore_barrier()

    @pl.when(s < NS - 1)
    def _():
        pltpu.sync_copy(acc.at[pl.ds(ROWS_PER_TILE * s, ROWS_PER_TILE)],
                        out_hbm.at[c, pl.ds(ROWS_PER_TILE * s, ROWS_PER_TILE)])

    @pl.when(s == NS - 1)
    def _():
        pltpu.sync_copy(acc.at[pl.ds(ROWS_PER_TILE * (NS - 1), LAST_ROWS)],
                        out_hbm.at[c, pl.ds(ROWS_PER_TILE * (NS - 1), LAST_ROWS)])


_BLK = 1000
_GRID = N // _BLK


def _tc0_body(x_ref, w1_ref, h1_ref):
    h1_ref[...] = jnp.dot(x_ref[...], w1_ref[...],
                          preferred_element_type=jnp.float32)


def _tc0(x, w1):
    return pl.pallas_call(
        _tc0_body,
        grid=(_GRID,),
        in_specs=[
            pl.BlockSpec((_BLK, D_IN), lambda i: (i, 0)),
            pl.BlockSpec((D_IN, D_HID), lambda i: (0, 0)),
        ],
        out_specs=pl.BlockSpec((_BLK, D_HID), lambda i: (i, 0)),
        out_shape=jax.ShapeDtypeStruct((N, D_HID), jnp.float32),
    )(x, w1)


def _tc1_body(h1_ref, d0_ref, d1_ref, h1s_ref, dinv_ref):
    dinv = lax.rsqrt(d0_ref[...] + d1_ref[...] + 1.0)
    h1s_ref[...] = h1_ref[...] * dinv
    dinv_ref[...] = dinv


def _tc1(h1, d0, d1):
    return pl.pallas_call(
        _tc1_body,
        grid=(_GRID,),
        in_specs=[
            pl.BlockSpec((_BLK, D_HID), lambda i: (i, 0)),
            pl.BlockSpec((_BLK, 1), lambda i: (i, 0)),
            pl.BlockSpec((_BLK, 1), lambda i: (i, 0)),
        ],
        out_specs=[
            pl.BlockSpec((_BLK, DW), lambda i: (i, 0)),
            pl.BlockSpec((_BLK, 1), lambda i: (i, 0)),
        ],
        out_shape=[
            jax.ShapeDtypeStruct((N, DW), jnp.float32),
            jax.ShapeDtypeStruct((N, 1), jnp.float32),
        ],
    )(h1, d0, d1)


def _tc2_body(p0_ref, p1_ref, h1s_ref, dinv_ref, b1_ref, w2_ref, h2s_ref):
    dinv = dinv_ref[...]
    t = p0_ref[0] + p1_ref[0] - h1s_ref[...]
    z1 = jnp.maximum(t * dinv + b1_ref[...], 0.0)
    h2 = jnp.dot(z1, w2_ref[...], preferred_element_type=jnp.float32)
    h2s_ref[...] = h2 * dinv


def _tc2(p, h1s, dinv, b1, w2):
    return pl.pallas_call(
        _tc2_body,
        grid=(_GRID,),
        in_specs=[
            pl.BlockSpec((1, _BLK, DW), lambda i: (0, i, 0)),
            pl.BlockSpec((1, _BLK, DW), lambda i: (1, i, 0)),
            pl.BlockSpec((_BLK, DW), lambda i: (i, 0)),
            pl.BlockSpec((_BLK, 1), lambda i: (i, 0)),
            pl.BlockSpec((1, D_HID), lambda i: (0, 0)),
            pl.BlockSpec((D_HID, D_HID), lambda i: (0, 0)),
        ],
        out_specs=pl.BlockSpec((_BLK, DW), lambda i: (i, 0)),
        out_shape=jax.ShapeDtypeStruct((N, DW), jnp.float32),
    )(p, p, h1s, dinv, b1, w2)


_BPAD = 24


def _tc3_body(q0_ref, q1_ref, h2s_ref, dinv_ref, b2_ref, batch_ref,
              out_ref, acc_ref):
    i = pl.program_id(0)
    z2 = jnp.maximum(
        (q0_ref[0] + q1_ref[0] - h2s_ref[...]) * dinv_ref[...]
        + b2_ref[...], 0.0)
    bb = batch_ref[0]
    iot = lax.broadcasted_iota(jnp.int32, (_BPAD, _BLK), 0)
    onehot = (iot == bb).astype(jnp.float32)
    part = jnp.dot(onehot, z2, preferred_element_type=jnp.float32)

    @pl.when(i == 0)
    def _():
        acc_ref[...] = part

    @pl.when(i > 0)
    def _():
        acc_ref[...] = acc_ref[...] + part

    @pl.when(i == _GRID - 1)
    def _():
        out_ref[...] = acc_ref[0:N_GRAPHS, :] * (1.0 / N_CLUSTERS)


def _tc3(q, h2s, dinv, b2, batch3d):
    return pl.pallas_call(
        _tc3_body,
        grid=(_GRID,),
        in_specs=[
            pl.BlockSpec((1, _BLK, DW), lambda i: (0, i, 0)),
            pl.BlockSpec((1, _BLK, DW), lambda i: (1, i, 0)),
            pl.BlockSpec((_BLK, DW), lambda i: (i, 0)),
            pl.BlockSpec((_BLK, 1), lambda i: (i, 0)),
            pl.BlockSpec((1, D_HID), lambda i: (0, 0)),
            pl.BlockSpec((1, 1, _BLK), lambda i: (i, 0, 0)),
        ],
        out_specs=pl.BlockSpec((N_GRAPHS, D_HID), lambda i: (0, 0)),
        out_shape=jax.ShapeDtypeStruct((N_GRAPHS, D_HID), jnp.float32),
        scratch_shapes=[pltpu.VMEM((_BPAD, D_HID), jnp.float32)],
    )(q, q, h2s, dinv, b2, batch3d)


def kernel(x, edge_index, batch, W1, b1, W2, b2, W3, b3, W4, b4):
    edges = edge_index.reshape(2, NW, NCH, CHUNK)

    h1 = _tc0(x, W1)
    deg = _sc_degree(edges)
    d0 = deg[0, 0, :N].reshape(N, 1)
    d1 = deg[1, 0, :N].reshape(N, 1)
    h1s, dinv = _tc1(h1, d0, d1)

    p = _sc_agg(h1s, edges)
    h2s = _tc2(p, h1s, dinv, b1.reshape(1, D_HID), W2)

    q = _sc_agg(h2s, edges)
    out = _tc3(q, h2s, dinv, b2.reshape(1, D_HID),
               batch.reshape(_GRID, 1, _BLK))
    return out
